# Initial kernel scaffold; baseline (speedup 1.0000x reference)
import jax
import jax.numpy as jnp
from jax.experimental import pallas as pl


def kernel(cart, numatoms, species, atom_index, shifts, W_emb, W1, b1, W2, b2, centers):
    raise NotImplementedError("write your pallas kernel here")


# speedup vs baseline: 7.7530x; 7.7530x over previous
import functools

import jax
import jax.numpy as jnp
from jax.experimental import pallas as pl

NMOL, MAXAT, NPAIR = 512, 48, 768
NRBF = 128
NSPECIES = 10
BLK = 8

_HIGH = jax.lax.Precision.HIGHEST


def _body(cart_ref, species_ref, idx_ref, shifts_ref, W_emb_ref, W1_ref,
          b1_ref, W2_ref, b2_ref, centers_ref, out_ref):
    W_emb = W_emb_ref[...]
    W1 = W1_ref[...]
    b1 = b1_ref[...]
    W2 = W2_ref[...]
    b2 = b2_ref[0, 0]
    centers = centers_ref[...]

    a48 = jax.lax.broadcasted_iota(jnp.int32, (NPAIR, MAXAT), 1)
    a48t = jax.lax.broadcasted_iota(jnp.int32, (MAXAT, NPAIR), 0)
    a10 = jax.lax.broadcasted_iota(jnp.int32, (MAXAT, NSPECIES), 1)

    for b in range(BLK):
        cart = cart_ref[b]
        spec = species_ref[b]
        idx_i = idx_ref[b, 0, :]
        idx_j = idx_ref[b, 1, :]
        sh = shifts_ref[b]

        Pi = (idx_i[:, None] == a48).astype(jnp.float32)
        Pj = (idx_j[:, None] == a48).astype(jnp.float32)
        PiT = (a48t == idx_i[None, :]).astype(jnp.float32)
        Ps = (spec[:, None] == a10).astype(jnp.float32)

        sel_i = jnp.dot(Pi, cart, precision=_HIGH)
        sel_j = jnp.dot(Pj, cart, precision=_HIGH)

        mask = jnp.all(sh > -1e10, axis=1).astype(jnp.float32)
        dvec = (sel_i - sel_j + sh) * mask[:, None]
        d2 = jnp.sum(dvec * dvec, axis=1)
        dist = jnp.sqrt(d2 + 1e-12)

        t = dist[:, None] - centers
        rbf = jnp.exp(-4.0 * t * t)

        emb = jnp.dot(Ps, W_emb, precision=_HIGH)
        embj = jnp.dot(Pj, emb, precision=_HIGH)
        contrib = rbf * embj * mask[:, None]

        dens = jnp.dot(PiT, contrib, precision=_HIGH)
        h = jnp.tanh(jnp.dot(dens, W1, precision=_HIGH) + b1)
        out = jnp.dot(h, W2, precision=_HIGH) + b2

        tot_vec = jnp.dot(PiT, dvec, precision=_HIGH)
        dipole = jnp.sum(out * tot_vec, axis=0, keepdims=True)
        out_ref[b, :] = dipole[0]


def kernel(cart, numatoms, species, atom_index, shifts, W_emb, W1, b1, W2,
           b2, centers):
    del numatoms
    nmol = cart.shape[0]
    idx = jnp.transpose(atom_index, (1, 0, 2)).astype(jnp.int32)

    grid = (nmol // BLK,)
    out = pl.pallas_call(
        _body,
        grid=grid,
        in_specs=[
            pl.BlockSpec((BLK, MAXAT, 3), lambda m: (m, 0, 0)),
            pl.BlockSpec((BLK, MAXAT), lambda m: (m, 0)),
            pl.BlockSpec((BLK, 2, NPAIR), lambda m: (m, 0, 0)),
            pl.BlockSpec((BLK, NPAIR, 3), lambda m: (m, 0, 0)),
            pl.BlockSpec((NSPECIES, NRBF), lambda m: (0, 0)),
            pl.BlockSpec((NRBF, 128), lambda m: (0, 0)),
            pl.BlockSpec((1, 128), lambda m: (0, 0)),
            pl.BlockSpec((128, 1), lambda m: (0, 0)),
            pl.BlockSpec((1, 1), lambda m: (0, 0)),
            pl.BlockSpec((1, NRBF), lambda m: (0, 0)),
        ],
        out_specs=pl.BlockSpec((BLK, 3), lambda m: (m, 0)),
        out_shape=jax.ShapeDtypeStruct((nmol, 3), jnp.float32),
    )(cart, species.astype(jnp.int32), idx, shifts, W_emb, W1,
      b1.reshape(1, 128), W2, b2.reshape(1, 1), centers.reshape(1, NRBF))
    return (out,)


# speedup vs baseline: 21.4832x; 2.7710x over previous
import jax
import jax.numpy as jnp
from jax.experimental import pallas as pl

NMOL, MAXAT, NPAIR = 512, 48, 768
NRBF = 128
NSPECIES = 10
BLK = 8

_HIGH = jax.lax.Precision.HIGHEST
_BF = jnp.bfloat16
_F32 = jnp.float32


def _mm(a, b):
    return jax.lax.dot_general(a, b, (((1,), (0,)), ((), ())),
                               preferred_element_type=_F32)


def _body(carthi_ref, cartlo_ref, species_ref, idxr_ref, idxi_ref,
          shiftsT_ref, W_embT_ref, W1T_ref, b1_ref, W2T_ref, b2_ref,
          centers_ref, out_ref):
    W_embT = W_embT_ref[...]
    W1T = W1T_ref[...]
    b1 = b1_ref[...]
    W2T = W2T_ref[...]
    b2 = b2_ref[0, 0]
    centers = centers_ref[...]
    neg4c2 = -4.0 * centers * centers

    iota_ap = jax.lax.broadcasted_iota(jnp.int32, (MAXAT, NPAIR), 0)
    iota_pa = jax.lax.broadcasted_iota(jnp.int32, (NPAIR, MAXAT), 1)
    iota_sa = jax.lax.broadcasted_iota(jnp.int32, (NSPECIES, MAXAT), 0)

    for b in range(BLK):
        carthi = carthi_ref[b]
        cartlo = cartlo_ref[b]
        spec = species_ref[b:b + 1, :]
        idx_i = idxr_ref[b, 0:1, :]
        idx_j = idxr_ref[b, 1:2, :]
        idx_i_col = idxi_ref[b]
        shT = shiftsT_ref[b]

        PiT = (iota_ap == idx_i).astype(_BF)
        PjT = (iota_ap == idx_j).astype(_BF)
        Pi = (iota_pa == idx_i_col).astype(_BF)
        PsT = (iota_sa == spec).astype(_BF)
        PdT = PiT - PjT

        dvecT = _mm(carthi, PdT) + _mm(cartlo, PdT) + shT

        m = shT > -1e10
        maskT = (m[0:1] & m[1:2] & m[2:3]).astype(_F32)

        d0 = dvecT[0:1, :]
        d1 = dvecT[1:2, :]
        d2row = dvecT[2:3, :]
        s2 = d0 * d0 + d1 * d1 + d2row * d2row + 1e-12
        dist = jnp.sqrt(s2)

        arg = centers * (8.0 * dist) + (neg4c2 + (-4.0) * s2)
        rbfT = jnp.exp(arg)

        SpecJ = _mm(PsT, PjT).astype(_BF)
        embjT = _mm(W_embT, SpecJ)
        contribT = (rbfT * embjT).astype(_BF)

        densT = _mm(contribT, Pi)
        hT = jnp.tanh(_mm(W1T, densT.astype(_BF)) + b1)
        outT = jnp.dot(W2T, hT, precision=_HIGH) + b2

        dvm = dvecT * maskT
        dvm_hi = dvm.astype(_BF)
        dvm_lo = (dvm - dvm_hi.astype(_F32)).astype(_BF)
        tot_vecT = _mm(dvm_hi, Pi) + _mm(dvm_lo, Pi)

        dipoleT = jnp.sum(tot_vecT * outT, axis=1, keepdims=True)
        out_ref[0, :, b:b + 1] = dipoleT


def kernel(cart, numatoms, species, atom_index, shifts, W_emb, W1, b1, W2,
           b2, centers):
    del numatoms
    nmol = cart.shape[0]
    cartT = jnp.transpose(cart, (0, 2, 1))
    cart_hi = cartT.astype(_BF)
    cart_lo = (cartT - cart_hi.astype(_F32)).astype(_BF)
    idx_rows = jnp.transpose(atom_index, (1, 0, 2)).astype(jnp.int32)
    idx_i_col = atom_index[0].astype(jnp.int32)[..., None]
    shiftsT = jnp.transpose(shifts, (0, 2, 1))

    grid = (nmol // BLK,)
    out = pl.pallas_call(
        _body,
        grid=grid,
        in_specs=[
            pl.BlockSpec((BLK, 3, MAXAT), lambda m: (m, 0, 0)),
            pl.BlockSpec((BLK, 3, MAXAT), lambda m: (m, 0, 0)),
            pl.BlockSpec((BLK, MAXAT), lambda m: (m, 0)),
            pl.BlockSpec((BLK, 2, NPAIR), lambda m: (m, 0, 0)),
            pl.BlockSpec((BLK, NPAIR, 1), lambda m: (m, 0, 0)),
            pl.BlockSpec((BLK, 3, NPAIR), lambda m: (m, 0, 0)),
            pl.BlockSpec((NRBF, NSPECIES), lambda m: (0, 0)),
            pl.BlockSpec((128, NRBF), lambda m: (0, 0)),
            pl.BlockSpec((128, 1), lambda m: (0, 0)),
            pl.BlockSpec((1, 128), lambda m: (0, 0)),
            pl.BlockSpec((1, 1), lambda m: (0, 0)),
            pl.BlockSpec((NRBF, 1), lambda m: (0, 0)),
        ],
        out_specs=pl.BlockSpec((1, 3, BLK), lambda m: (m, 0, 0)),
        out_shape=jax.ShapeDtypeStruct((nmol // BLK, 3, BLK), jnp.float32),
    )(cart_hi, cart_lo, species.astype(jnp.int32), idx_rows, idx_i_col,
      shiftsT, W_emb.T.astype(_BF), W1.T.astype(_BF), b1.reshape(128, 1),
      W2.reshape(1, 128), b2.reshape(1, 1), centers.reshape(NRBF, 1))
    return (jnp.transpose(out, (0, 2, 1)).reshape(nmol, 3),)


# speedup vs baseline: 25.9733x; 1.2090x over previous
import jax
import jax.numpy as jnp
from jax import lax
from jax.experimental import pallas as pl
from jax.experimental.pallas import tpu as pltpu
from jax.experimental.pallas import tpu_sc as plsc

NMOL, MAXAT, NPAIR = 512, 48, 768
NRBF = 128
NSPECIES = 10
BLK = 8
NWORKERS = 32
MPW = NMOL // NWORKERS

_HIGH = jax.lax.Precision.HIGHEST
_BF = jnp.bfloat16
_F32 = jnp.float32


def _mm(a, b):
    return jax.lax.dot_general(a, b, (((1,), (0,)), ((), ())),
                               preferred_element_type=_F32)


def _sc_body(cart_hbm, idx_hbm, shiftsT_hbm, species_hbm,
             s2_hbm, specj_hbm, tot_hbm,
             cart_v, idx_v, sh_v, spec_v, s2_v, sj_v, tot_v):
    wid = lax.axis_index("s") * 2 + lax.axis_index("c")

    zf = jnp.zeros((16,), _F32)

    def per_mol(t, carry):
        m = wid * MPW + t
        pltpu.sync_copy(cart_hbm.at[m], cart_v)
        pltpu.sync_copy(idx_hbm.at[m], idx_v)
        pltpu.sync_copy(shiftsT_hbm.at[m], sh_v)
        pltpu.sync_copy(species_hbm.at[m], spec_v)
        for c in range(3 * MAXAT // 16):
            tot_v[pl.ds(c * 16, 16)] = zf
        for p in range(0, NPAIR, 16):
            sl = pl.ds(p, 16)
            ivec = idx_v[0, sl]
            jvec = idx_v[1, sl]
            i1 = ivec + MAXAT
            i2 = ivec + 2 * MAXAT
            ci0 = plsc.load_gather(cart_v, [ivec])
            ci1 = plsc.load_gather(cart_v, [i1])
            ci2 = plsc.load_gather(cart_v, [i2])
            cj0 = plsc.load_gather(cart_v, [jvec])
            cj1 = plsc.load_gather(cart_v, [jvec + MAXAT])
            cj2 = plsc.load_gather(cart_v, [jvec + 2 * MAXAT])
            sh0 = sh_v[0, sl]
            sh1 = sh_v[1, sl]
            sh2 = sh_v[2, sl]
            dv0 = ci0 - cj0 + sh0
            dv1 = ci1 - cj1 + sh1
            dv2 = ci2 - cj2 + sh2
            s2_v[sl] = dv0 * dv0 + dv1 * dv1 + dv2 * dv2
            valid = (sh0 > -1e10) & (sh1 > -1e10) & (sh2 > -1e10)
            plsc.addupdate_scatter(tot_v, [ivec],
                                   jnp.where(valid, dv0, 0.0))
            plsc.addupdate_scatter(tot_v, [i1],
                                   jnp.where(valid, dv1, 0.0))
            plsc.addupdate_scatter(tot_v, [i2],
                                   jnp.where(valid, dv2, 0.0))
            sj_v[sl] = plsc.load_gather(spec_v, [jvec])
        pltpu.sync_copy(s2_v, s2_hbm.at[m])
        pltpu.sync_copy(sj_v, specj_hbm.at[m])
        pltpu.sync_copy(tot_v, tot_hbm.at[m])
        return carry

    lax.fori_loop(0, MPW, per_mol, 0)


def _run_sc(cartT, idx_rows, shiftsT, species):
    fn = pl.kernel(
        _sc_body,
        out_type=[
            jax.ShapeDtypeStruct((NMOL, NPAIR), _F32),
            jax.ShapeDtypeStruct((NMOL, NPAIR), jnp.int32),
            jax.ShapeDtypeStruct((NMOL, 3 * MAXAT), _F32),
        ],
        mesh=plsc.VectorSubcoreMesh(core_axis_name="c", subcore_axis_name="s"),
        compiler_params=pltpu.CompilerParams(needs_layout_passes=False),
        scratch_types=[
            pltpu.VMEM((3 * MAXAT,), _F32),
            pltpu.VMEM((2, NPAIR), jnp.int32),
            pltpu.VMEM((3, NPAIR), _F32),
            pltpu.VMEM((MAXAT,), jnp.int32),
            pltpu.VMEM((NPAIR,), _F32),
            pltpu.VMEM((NPAIR,), jnp.int32),
            pltpu.VMEM((3 * MAXAT,), _F32),
        ],
    )
    return fn(cartT, idx_rows, shiftsT, species)


def _tc_body(s2_ref, specj_ref, tot_ref, idxi_ref, W_embT_ref, W1T_ref,
             b1_ref, W2T_ref, b2_ref, centers_ref, out_ref):
    W_embT = W_embT_ref[...]
    W1T = W1T_ref[...]
    b1 = b1_ref[...]
    W2T = W2T_ref[...]
    b2 = b2_ref[0, 0]
    centers = centers_ref[...]
    neg4c2 = -4.0 * centers * centers

    iota_pa = jax.lax.broadcasted_iota(jnp.int32, (NPAIR, MAXAT), 1)
    iota_sp = jax.lax.broadcasted_iota(jnp.int32, (NSPECIES, NPAIR), 0)

    for b in range(BLK):
        s2row = s2_ref[b:b + 1, :] + 1e-12
        dist = jnp.sqrt(s2row)
        arg = centers * (8.0 * dist) + (neg4c2 + (-4.0) * s2row)
        rbfT = jnp.exp(arg)

        specj = specj_ref[b:b + 1, :]
        SpecJ = (iota_sp == specj).astype(_BF)
        embjT = _mm(W_embT, SpecJ)
        contribT = (rbfT * embjT).astype(_BF)

        Pi = (iota_pa == idxi_ref[b]).astype(_BF)
        densT = _mm(contribT, Pi)
        hT = jnp.tanh(_mm(W1T, densT.astype(_BF)) + b1)
        outT = jnp.dot(W2T, hT, precision=_HIGH) + b2

        totT = tot_ref[b]
        dipoleT = jnp.sum(totT * outT, axis=1, keepdims=True)
        out_ref[0, :, b:b + 1] = dipoleT


def kernel(cart, numatoms, species, atom_index, shifts, W_emb, W1, b1, W2,
           b2, centers):
    del numatoms
    nmol = cart.shape[0]
    cartT = jnp.transpose(cart, (0, 2, 1)).reshape(nmol, 3 * MAXAT)
    idx_rows = jnp.transpose(atom_index, (1, 0, 2)).astype(jnp.int32)
    idx_i_col = atom_index[0].astype(jnp.int32)[..., None]
    shiftsT = jnp.transpose(shifts, (0, 2, 1))
    species32 = species.astype(jnp.int32)

    s2, specj, tot_vec = _run_sc(cartT, idx_rows, shiftsT, species32)
    tot_vec = tot_vec.reshape(nmol, 3, MAXAT)

    grid = (nmol // BLK,)
    out = pl.pallas_call(
        _tc_body,
        grid=grid,
        in_specs=[
            pl.BlockSpec((BLK, NPAIR), lambda m: (m, 0)),
            pl.BlockSpec((BLK, NPAIR), lambda m: (m, 0)),
            pl.BlockSpec((BLK, 3, MAXAT), lambda m: (m, 0, 0)),
            pl.BlockSpec((BLK, NPAIR, 1), lambda m: (m, 0, 0)),
            pl.BlockSpec((NRBF, NSPECIES), lambda m: (0, 0)),
            pl.BlockSpec((128, NRBF), lambda m: (0, 0)),
            pl.BlockSpec((128, 1), lambda m: (0, 0)),
            pl.BlockSpec((1, 128), lambda m: (0, 0)),
            pl.BlockSpec((1, 1), lambda m: (0, 0)),
            pl.BlockSpec((NRBF, 1), lambda m: (0, 0)),
        ],
        out_specs=pl.BlockSpec((1, 3, BLK), lambda m: (m, 0, 0)),
        out_shape=jax.ShapeDtypeStruct((nmol // BLK, 3, BLK), jnp.float32),
    )(s2, specj, tot_vec, idx_i_col, W_emb.T.astype(_BF), W1.T.astype(_BF),
      b1.reshape(128, 1), W2.reshape(1, 128), b2.reshape(1, 1),
      centers.reshape(NRBF, 1))
    return (jnp.transpose(out, (0, 2, 1)).reshape(nmol, 3),)


# speedup vs baseline: 25.9804x; 1.0003x over previous
import jax
import jax.numpy as jnp
from jax import lax
from jax.experimental import pallas as pl
from jax.experimental.pallas import tpu as pltpu
from jax.experimental.pallas import tpu_sc as plsc

NMOL, MAXAT, NPAIR = 512, 48, 768
NRBF = 128
NSPECIES = 10
BLK = 8
NWORKERS = 32
MPW = NMOL // NWORKERS

_HIGH = jax.lax.Precision.HIGHEST
_BF = jnp.bfloat16
_F32 = jnp.float32


def _mm(a, b):
    return jax.lax.dot_general(a, b, (((1,), (0,)), ((), ())),
                               preferred_element_type=_F32)


def _sc_body(cart_hbm, idx_hbm, shiftsT_hbm, species_hbm,
             s2_hbm, specj_hbm, tot_hbm,
             cart_v0, idx_v0, sh_v0, spec_v0, s2_v0, sj_v0, tot_v0,
             cart_v1, idx_v1, sh_v1, spec_v1, s2_v1, sj_v1, tot_v1,
             in_sem0, in_sem1, out_sem0, out_sem1):
    wid = lax.axis_index("s") * 2 + lax.axis_index("c")
    base = wid * MPW

    slots = (
        (cart_v0, idx_v0, sh_v0, spec_v0, s2_v0, sj_v0, tot_v0,
         in_sem0, out_sem0),
        (cart_v1, idx_v1, sh_v1, spec_v1, s2_v1, sj_v1, tot_v1,
         in_sem1, out_sem1),
    )
    zf = jnp.zeros((16,), _F32)

    def issue_in(m, slot):
        cart_v, idx_v, sh_v, spec_v, _, _, _, in_sem, _ = slot
        pltpu.async_copy(cart_hbm.at[m], cart_v, in_sem)
        pltpu.async_copy(idx_hbm.at[m], idx_v, in_sem)
        pltpu.async_copy(shiftsT_hbm.at[m], sh_v, in_sem)
        pltpu.async_copy(species_hbm.at[m], spec_v, in_sem)

    def wait_in(m, slot):
        cart_v, idx_v, sh_v, spec_v, _, _, _, in_sem, _ = slot
        pltpu.make_async_copy(cart_hbm.at[m], cart_v, in_sem).wait()
        pltpu.make_async_copy(idx_hbm.at[m], idx_v, in_sem).wait()
        pltpu.make_async_copy(shiftsT_hbm.at[m], sh_v, in_sem).wait()
        pltpu.make_async_copy(species_hbm.at[m], spec_v, in_sem).wait()

    issue_in(base, slots[0])
    issue_in(base + 1, slots[1])

    def per_pair(t2, carry):
        for s in range(2):
            slot = slots[s]
            cart_v, idx_v, sh_v, spec_v, s2_v, sj_v, tot_v, in_sem, \
                out_sem = slot
            m = base + 2 * t2 + s
            wait_in(m, slot)
            for c in range(3 * MAXAT // 16):
                tot_v[pl.ds(c * 16, 16)] = zf
            for p in range(0, NPAIR, 16):
                sl = pl.ds(p, 16)
                ivec = idx_v[0, sl]
                jvec = idx_v[1, sl]
                i1 = ivec + MAXAT
                i2 = ivec + 2 * MAXAT
                ci0 = plsc.load_gather(cart_v, [ivec])
                ci1 = plsc.load_gather(cart_v, [i1])
                ci2 = plsc.load_gather(cart_v, [i2])
                cj0 = plsc.load_gather(cart_v, [jvec])
                cj1 = plsc.load_gather(cart_v, [jvec + MAXAT])
                cj2 = plsc.load_gather(cart_v, [jvec + 2 * MAXAT])
                sh0 = sh_v[0, sl]
                sh1 = sh_v[1, sl]
                sh2 = sh_v[2, sl]
                dv0 = ci0 - cj0 + sh0
                dv1 = ci1 - cj1 + sh1
                dv2 = ci2 - cj2 + sh2
                s2_v[sl] = dv0 * dv0 + dv1 * dv1 + dv2 * dv2
                valid = (sh0 > -1e10) & (sh1 > -1e10) & (sh2 > -1e10)
                plsc.addupdate_scatter(tot_v, [ivec],
                                       jnp.where(valid, dv0, 0.0))
                plsc.addupdate_scatter(tot_v, [i1],
                                       jnp.where(valid, dv1, 0.0))
                plsc.addupdate_scatter(tot_v, [i2],
                                       jnp.where(valid, dv2, 0.0))
                sj_v[sl] = plsc.load_gather(spec_v, [jvec])
            issue_in(jnp.minimum(m + 2, NMOL - 1), slot)
            out_s2 = pltpu.make_async_copy(s2_v, s2_hbm.at[m], out_sem)
            out_sj = pltpu.make_async_copy(sj_v, specj_hbm.at[m], out_sem)
            out_tot = pltpu.make_async_copy(tot_v, tot_hbm.at[m], out_sem)
            out_s2.start()
            out_sj.start()
            out_tot.start()
            out_s2.wait()
            out_sj.wait()
            out_tot.wait()
        return carry

    lax.fori_loop(0, MPW // 2, per_pair, 0)
    wait_in(base, slots[0])
    wait_in(base, slots[1])


def _run_sc(cartT, idx_rows, shiftsT, species):
    fn = pl.kernel(
        _sc_body,
        out_type=[
            jax.ShapeDtypeStruct((NMOL, NPAIR), _F32),
            jax.ShapeDtypeStruct((NMOL, NPAIR), jnp.int32),
            jax.ShapeDtypeStruct((NMOL, 3 * MAXAT), _F32),
        ],
        mesh=plsc.VectorSubcoreMesh(core_axis_name="c", subcore_axis_name="s"),
        compiler_params=pltpu.CompilerParams(needs_layout_passes=False),
        scratch_types=(
            [pltpu.VMEM((3 * MAXAT,), _F32),
             pltpu.VMEM((2, NPAIR), jnp.int32),
             pltpu.VMEM((3, NPAIR), _F32),
             pltpu.VMEM((MAXAT,), jnp.int32),
             pltpu.VMEM((NPAIR,), _F32),
             pltpu.VMEM((NPAIR,), jnp.int32),
             pltpu.VMEM((3 * MAXAT,), _F32)] * 2
            + [pltpu.SemaphoreType.DMA] * 4
        ),
    )
    return fn(cartT, idx_rows, shiftsT, species)


def _tc_body(s2_ref, specj_ref, tot_ref, idxi_ref, W_embT_ref, W1T_ref,
             b1_ref, W2T_ref, b2_ref, centers_ref, out_ref):
    W_embT = W_embT_ref[...]
    W1T = W1T_ref[...]
    b1 = b1_ref[...]
    W2T = W2T_ref[...]
    b2 = b2_ref[0, 0]
    centers = centers_ref[...]
    neg4c2 = -4.0 * centers * centers

    iota_pa = jax.lax.broadcasted_iota(jnp.int32, (NPAIR, MAXAT), 1)
    iota_sp = jax.lax.broadcasted_iota(jnp.int32, (NSPECIES, NPAIR), 0)

    for b in range(BLK):
        s2row = s2_ref[b:b + 1, :] + 1e-12
        dist = jnp.sqrt(s2row)
        arg = centers * (8.0 * dist) + (neg4c2 + (-4.0) * s2row)
        rbfT = jnp.exp(arg)

        specj = specj_ref[b:b + 1, :]
        SpecJ = (iota_sp == specj).astype(_BF)
        embjT = _mm(W_embT, SpecJ)
        contribT = (rbfT * embjT).astype(_BF)

        Pi = (iota_pa == idxi_ref[b]).astype(_BF)
        densT = _mm(contribT, Pi)
        hT = jnp.tanh(_mm(W1T, densT.astype(_BF)) + b1)
        outT = jnp.dot(W2T, hT, precision=_HIGH) + b2

        totT = tot_ref[b]
        dipoleT = jnp.sum(totT * outT, axis=1, keepdims=True)
        out_ref[0, :, b:b + 1] = dipoleT


def kernel(cart, numatoms, species, atom_index, shifts, W_emb, W1, b1, W2,
           b2, centers):
    del numatoms
    nmol = cart.shape[0]
    cartT = jnp.transpose(cart, (0, 2, 1)).reshape(nmol, 3 * MAXAT)
    idx_rows = jnp.transpose(atom_index, (1, 0, 2)).astype(jnp.int32)
    idx_i_col = atom_index[0].astype(jnp.int32)[..., None]
    shiftsT = jnp.transpose(shifts, (0, 2, 1))
    species32 = species.astype(jnp.int32)

    s2, specj, tot_vec = _run_sc(cartT, idx_rows, shiftsT, species32)
    tot_vec = tot_vec.reshape(nmol, 3, MAXAT)

    grid = (nmol // BLK,)
    out = pl.pallas_call(
        _tc_body,
        grid=grid,
        in_specs=[
            pl.BlockSpec((BLK, NPAIR), lambda m: (m, 0)),
            pl.BlockSpec((BLK, NPAIR), lambda m: (m, 0)),
            pl.BlockSpec((BLK, 3, MAXAT), lambda m: (m, 0, 0)),
            pl.BlockSpec((BLK, NPAIR, 1), lambda m: (m, 0, 0)),
            pl.BlockSpec((NRBF, NSPECIES), lambda m: (0, 0)),
            pl.BlockSpec((128, NRBF), lambda m: (0, 0)),
            pl.BlockSpec((128, 1), lambda m: (0, 0)),
            pl.BlockSpec((1, 128), lambda m: (0, 0)),
            pl.BlockSpec((1, 1), lambda m: (0, 0)),
            pl.BlockSpec((NRBF, 1), lambda m: (0, 0)),
        ],
        out_specs=pl.BlockSpec((1, 3, BLK), lambda m: (m, 0, 0)),
        out_shape=jax.ShapeDtypeStruct((nmol // BLK, 3, BLK), jnp.float32),
    )(s2, specj, tot_vec, idx_i_col, W_emb.T.astype(_BF), W1.T.astype(_BF),
      b1.reshape(128, 1), W2.reshape(1, 128), b2.reshape(1, 1),
      centers.reshape(NRBF, 1))
    return (jnp.transpose(out, (0, 2, 1)).reshape(nmol, 3),)
